# baseline (speedup 1.0000x reference)
import jax
import jax.numpy as jnp
from jax import lax
from jax.experimental import pallas as pl
from jax.experimental.pallas import tpu as pltpu
from jax.experimental.pallas import tpu_sc as plsc

NUM_CORES = 2
NUM_SUBCORES = 16
NUM_WORKERS = NUM_CORES * NUM_SUBCORES
CHUNK_B = 32
BLK_B = 32


def _gather_kernel(table_hbm, ids_hbm, out_hbm, idx2_v, idx_v, rows_v, sem):
    n_batch, seq = ids_hbm.shape
    b_per_w = n_batch // NUM_WORKERS
    wid = lax.axis_index("s") * NUM_CORES + lax.axis_index("c")
    b0w = wid * b_per_w
    n_ids = CHUNK_B * seq
    lane = lax.iota(jnp.int32, 16)

    @pl.loop(0, b_per_w, step=CHUNK_B)
    def _(bo):
        b0 = b0w + bo
        pltpu.sync_copy(ids_hbm.at[pl.ds(b0, CHUNK_B)], idx2_v)
        for j in range(n_ids // 16):
            flat = lane + (16 * j)
            vals = plsc.load_gather(idx2_v, [flat // seq, flat % seq])
            idx_v[pl.ds(16 * j, 16)] = vals >> 1
        pltpu.async_copy(table_hbm.at[idx_v], rows_v, sem).wait()
        pltpu.sync_copy(rows_v, out_hbm.at[pl.ds(b0 * seq, n_ids)])


def _select_kernel(pairs_ref, ids_ref, out_ref):
    blk_b, seq, dim = out_ref.shape
    x = pairs_ref[...].reshape(blk_b, seq, 2 * dim)
    h = (ids_ref[...] & 1)[:, :, None]
    out_ref[...] = jnp.where(h == 1, x[:, :, dim:], x[:, :, :dim])


def kernel(token_ids, embedding_table):
    batch, seq = token_ids.shape
    n_rows, dim = embedding_table.shape
    ids = token_ids.astype(jnp.int32)
    table_pairs = embedding_table.reshape(n_rows // 2, 2 * dim)

    mesh = plsc.VectorSubcoreMesh(core_axis_name="c", subcore_axis_name="s")
    gather = pl.kernel(
        _gather_kernel,
        mesh=mesh,
        out_type=jax.ShapeDtypeStruct((batch * seq, 2 * dim), jnp.float32),
        scratch_types=[
            pltpu.VMEM((CHUNK_B, seq), jnp.int32),
            pltpu.VMEM((CHUNK_B * seq,), jnp.int32),
            pltpu.VMEM((CHUNK_B * seq, 2 * dim), jnp.float32),
            pltpu.SemaphoreType.DMA,
        ],
        compiler_params=pltpu.CompilerParams(needs_layout_passes=False),
    )
    inter = gather(table_pairs, ids)

    select = pl.pallas_call(
        _select_kernel,
        out_shape=jax.ShapeDtypeStruct((batch, seq, dim), jnp.float32),
        grid=(batch // BLK_B,),
        in_specs=[
            pl.BlockSpec((BLK_B * seq, 2 * dim), lambda i: (i, 0)),
            pl.BlockSpec((BLK_B, seq), lambda i: (i, 0)),
        ],
        out_specs=pl.BlockSpec((BLK_B, seq, dim), lambda i: (i, 0, 0)),
    )
    return select(inter, ids)


# speedup vs baseline: 1.0447x; 1.0447x over previous
import jax
import jax.numpy as jnp
from jax import lax
from jax.experimental import pallas as pl
from jax.experimental.pallas import tpu as pltpu
from jax.experimental.pallas import tpu_sc as plsc

NUM_CORES = 2
NUM_SUBCORES = 16
NUM_WORKERS = NUM_CORES * NUM_SUBCORES
CHUNK_B = 32


def _gather_kernel(table_hbm, ids_hbm, out_hbm, idx_v, rows_v, sem):
    n_chunks = ids_hbm.shape[0]
    n_ids = ids_hbm.shape[2]
    dim = table_hbm.shape[1]
    half_rows = out_hbm.shape[0]
    c_per_w = n_chunks // NUM_WORKERS
    wid = lax.axis_index("s") * NUM_CORES + lax.axis_index("c")
    c0 = wid * c_per_w

    @pl.loop(0, c_per_w)
    def _(k):
        chunk = c0 + k
        pltpu.sync_copy(ids_hbm.at[chunk, 0], idx_v)
        pltpu.async_copy(table_hbm.at[idx_v], rows_v, sem).wait()
        t0 = chunk * n_ids
        row0 = lax.rem(t0, half_rows)
        col0 = lax.div(t0, half_rows) * dim
        pltpu.sync_copy(
            rows_v, out_hbm.at[pl.ds(row0, n_ids), pl.ds(col0, dim)]
        )


def _select_kernel(pairs_ref, out_ref):
    blk_b, seq, dim = out_ref.shape
    x = pairs_ref[...].reshape(blk_b, seq, 2 * dim)
    on_right = pl.program_id(1) == 1
    out_ref[...] = jnp.where(on_right, x[:, :, dim:], x[:, :, :dim])


def kernel(token_ids, embedding_table):
    batch, seq = token_ids.shape
    dim = embedding_table.shape[1]
    n_flat = batch * seq
    n_chunks = n_flat // (CHUNK_B * seq)
    ids3 = token_ids.astype(jnp.int32).reshape(n_chunks, 1, CHUNK_B * seq)

    mesh = plsc.VectorSubcoreMesh(core_axis_name="c", subcore_axis_name="s")
    gather = pl.kernel(
        _gather_kernel,
        mesh=mesh,
        out_type=jax.ShapeDtypeStruct((n_flat // 2, 2 * dim), jnp.float32),
        scratch_types=[
            pltpu.VMEM((CHUNK_B * seq,), jnp.int32),
            pltpu.VMEM((CHUNK_B * seq, dim), jnp.float32),
            pltpu.SemaphoreType.DMA,
        ],
        compiler_params=pltpu.CompilerParams(
            use_tc_tiling_on_sc=False, needs_layout_passes=False
        ),
    )
    inter = gather(embedding_table, ids3)

    half_blocks = n_chunks // 2
    select = pl.pallas_call(
        _select_kernel,
        out_shape=jax.ShapeDtypeStruct((batch, seq, dim), jnp.float32),
        grid=(half_blocks, 2),
        in_specs=[
            pl.BlockSpec((CHUNK_B * seq, 2 * dim), lambda i, h: (i, 0)),
        ],
        out_specs=pl.BlockSpec(
            (CHUNK_B, seq, dim), lambda i, h: (h * half_blocks + i, 0, 0)
        ),
    )
    return select(inter)
